# Initial kernel scaffold; baseline (speedup 1.0000x reference)
import jax
import jax.numpy as jnp
from jax.experimental import pallas as pl


def kernel(x, edges, params):
    raise NotImplementedError("write your pallas kernel here")


# speedup vs baseline: 1.0001x; 1.0001x over previous
import jax
import jax.numpy as jnp
from jax.experimental import pallas as pl
from jax.experimental.pallas import tpu as pltpu

DIM = 256
N_NODES = 10000
N_EDGES = 160000
EPS = 1e-5


def _final_stage_kernel(xn_ref, wt_ref, b_ref, g_ref, bb_ref, out_ref):
    y = jnp.dot(xn_ref[...], wt_ref[...], preferred_element_type=jnp.float32)
    y = y + b_ref[...]
    m = jnp.mean(y, axis=0, keepdims=True)
    v = jnp.mean((y - m) ** 2, axis=0, keepdims=True)
    out_ref[...] = jnp.maximum((y - m) / jnp.sqrt(v + EPS) * g_ref[...] + bb_ref[...], 0.0)


def _final_stage(x_new, W, b, g, bb):
    return pl.pallas_call(
        _final_stage_kernel,
        out_shape=jax.ShapeDtypeStruct((N_NODES, DIM), jnp.float32),
    )(x_new, W.T, b.reshape(1, DIM), g.reshape(1, DIM), bb.reshape(1, DIM))


def _bn(x, g, b):
    m = jnp.mean(x, axis=0)
    v = jnp.var(x, axis=0)
    return (x - m) / jnp.sqrt(v + EPS) * g + b


def _block2(h, p, pre):
    h = jax.nn.relu(_bn(h, p[pre + '_bn1_g'], p[pre + '_bn1_b']))
    h = jax.nn.relu(_bn(h @ p[pre + '_fc2_W'].T + p[pre + '_fc2_b'],
                        p[pre + '_bn2_g'], p[pre + '_bn2_b']))
    return h


def kernel(x, edges, params):
    p = params
    FI_a = x @ p['FI_fc1a_W'].T
    FI_b = x @ p['FI_fc1b_W'].T + p['FI_fc1b_b']
    FO_a = x @ p['FO_fc1a_W'].T
    FO_b = x @ p['FO_fc1b_W'].T + p['FO_fc1b_b']
    src = edges[:, 0]
    dst = edges[:, 1]
    FI_in = FI_a[src] + FI_b[dst]
    FO_in = FO_a[src] + FO_b[dst]
    FI_out = _block2(FI_in, p, 'FI')
    FO_out = _block2(FO_in, p, 'FO')
    d = (jax.ops.segment_sum(FI_out, dst, num_segments=N_NODES)
         + jax.ops.segment_sum(FO_out, src, num_segments=N_NODES))
    x_new = x + d
    return _final_stage(x_new, p['FP_fc_W'], p['FP_fc_b'], p['FP_bn_g'], p['FP_bn_b'])


# speedup vs baseline: 1.5357x; 1.5355x over previous
import functools

import jax
import jax.numpy as jnp
from jax import lax
from jax.experimental import pallas as pl
from jax.experimental.pallas import tpu as pltpu
from jax.experimental.pallas import tpu_sc as plsc

DIM = 256
N_NODES = 10000
N_EDGES = 160000
EPS = 1e-5

_SC_INFO = plsc.get_sparse_core_info()
_NC, _NS, _L = _SC_INFO.num_cores, _SC_INFO.num_subcores, _SC_INFO.num_lanes
_NW = _NC * _NS

_EPW = N_EDGES // _NW
_KB = 200
_NCHUNK = _EPW // _KB

_sc_mesh = functools.partial(
    plsc.VectorSubcoreMesh, core_axis_name="c", subcore_axis_name="s")


def _gather_add_body(fia, fib, foa, fob, src_hbm, dst_hbm, fi_out, fo_out,
                     idx_s, idx_d, bufa, bufb, sem_a, sem_b):
    wid = lax.axis_index("s") * _NC + lax.axis_index("c")
    base0 = wid * _EPW

    def chunk(j, carry):
        base = base0 + j * _KB
        pltpu.sync_copy(src_hbm.at[pl.ds(base, _KB)], idx_s)
        pltpu.sync_copy(dst_hbm.at[pl.ds(base, _KB)], idx_d)
        for ta, tb, out in ((fia, fib, fi_out), (foa, fob, fo_out)):
            cp_a = pltpu.async_copy(ta.at[idx_s], bufa, sem_a)
            cp_b = pltpu.async_copy(tb.at[idx_d], bufb, sem_b)
            cp_a.wait()
            cp_b.wait()

            def row(i, c):
                for g in range(DIM // _L):
                    sl = pl.ds(g * _L, _L)
                    bufa[i, sl] = bufa[i, sl] + bufb[i, sl]
                return c

            lax.fori_loop(0, _KB, row, 0, unroll=2)
            pltpu.sync_copy(bufa, out.at[pl.ds(base, _KB)])
        return carry

    lax.fori_loop(0, _NCHUNK, chunk, 0)


@functools.partial(jax.jit, donate_argnums=())
def _gather_add(fia, fib, foa, fob, src, dst):
    f = pl.kernel(
        _gather_add_body,
        out_type=[jax.ShapeDtypeStruct((N_EDGES, DIM), jnp.float32),
                  jax.ShapeDtypeStruct((N_EDGES, DIM), jnp.float32)],
        mesh=_sc_mesh(),
        scratch_types=[
            pltpu.VMEM((_KB,), jnp.int32),
            pltpu.VMEM((_KB,), jnp.int32),
            pltpu.VMEM((_KB, DIM), jnp.float32),
            pltpu.VMEM((_KB, DIM), jnp.float32),
            pltpu.SemaphoreType.DMA,
            pltpu.SemaphoreType.DMA,
        ],
    )
    return f(fia, fib, foa, fob, src, dst)


def _final_stage_kernel(xn_ref, wt_ref, b_ref, g_ref, bb_ref, out_ref):
    y = jnp.dot(xn_ref[...], wt_ref[...], preferred_element_type=jnp.float32)
    y = y + b_ref[...]
    m = jnp.mean(y, axis=0, keepdims=True)
    v = jnp.mean((y - m) ** 2, axis=0, keepdims=True)
    out_ref[...] = jnp.maximum((y - m) / jnp.sqrt(v + EPS) * g_ref[...] + bb_ref[...], 0.0)


def _final_stage(x_new, W, b, g, bb):
    return pl.pallas_call(
        _final_stage_kernel,
        out_shape=jax.ShapeDtypeStruct((N_NODES, DIM), jnp.float32),
    )(x_new, W.T, b.reshape(1, DIM), g.reshape(1, DIM), bb.reshape(1, DIM))


def _bn(x, g, b):
    m = jnp.mean(x, axis=0)
    v = jnp.var(x, axis=0)
    return (x - m) / jnp.sqrt(v + EPS) * g + b


def _block2(h, p, pre):
    h = jax.nn.relu(_bn(h, p[pre + '_bn1_g'], p[pre + '_bn1_b']))
    h = jax.nn.relu(_bn(h @ p[pre + '_fc2_W'].T + p[pre + '_fc2_b'],
                        p[pre + '_bn2_g'], p[pre + '_bn2_b']))
    return h


def kernel(x, edges, params):
    p = params
    FI_a = x @ p['FI_fc1a_W'].T
    FI_b = x @ p['FI_fc1b_W'].T + p['FI_fc1b_b']
    FO_a = x @ p['FO_fc1a_W'].T
    FO_b = x @ p['FO_fc1b_W'].T + p['FO_fc1b_b']
    src = jnp.asarray(edges[:, 0], jnp.int32)
    dst = jnp.asarray(edges[:, 1], jnp.int32)
    FI_in, FO_in = _gather_add(FI_a, FI_b, FO_a, FO_b, src, dst)
    FI_out = _block2(FI_in, p, 'FI')
    FO_out = _block2(FO_in, p, 'FO')
    d = (jax.ops.segment_sum(FI_out, dst, num_segments=N_NODES)
         + jax.ops.segment_sum(FO_out, src, num_segments=N_NODES))
    x_new = x + d
    return _final_stage(x_new, p['FP_fc_W'], p['FP_fc_b'], p['FP_bn_g'], p['FP_bn_b'])


# speedup vs baseline: 2.2345x; 1.4551x over previous
import functools

import jax
import jax.numpy as jnp
from jax import lax
from jax.experimental import pallas as pl
from jax.experimental.pallas import tpu as pltpu
from jax.experimental.pallas import tpu_sc as plsc

DIM = 256
N_NODES = 10000
N_EDGES = 160000
EPS = 1e-5

_SC_INFO = plsc.get_sparse_core_info()
_NC, _NS, _L = _SC_INFO.num_cores, _SC_INFO.num_subcores, _SC_INFO.num_lanes
_NW = _NC * _NS

_sc_mesh = functools.partial(
    plsc.VectorSubcoreMesh, core_axis_name="c", subcore_axis_name="s")


_EPW = N_EDGES // _NW
_KB = 200
_NCHUNK = _EPW // _KB


def _gather_add_body(ta, tb, src_hbm, dst_hbm, out,
                     idx_s, idx_d, bufa, bufb, sem_a, sem_b):
    wid = lax.axis_index("s") * _NC + lax.axis_index("c")
    base0 = wid * _EPW

    def chunk(j, carry):
        base = base0 + j * _KB
        pltpu.sync_copy(src_hbm.at[pl.ds(base, _KB)], idx_s)
        pltpu.sync_copy(dst_hbm.at[pl.ds(base, _KB)], idx_d)
        cp_a = pltpu.async_copy(ta.at[idx_s], bufa, sem_a)
        cp_b = pltpu.async_copy(tb.at[idx_d], bufb, sem_b)
        cp_a.wait()
        cp_b.wait()

        def row(i, c):
            for g in range(DIM // _L):
                sl = pl.ds(g * _L, _L)
                bufa[i, sl] = bufa[i, sl] + bufb[i, sl]
            return c

        lax.fori_loop(0, _KB, row, 0, unroll=2)
        pltpu.sync_copy(bufa, out.at[pl.ds(base, _KB)])
        return carry

    lax.fori_loop(0, _NCHUNK, chunk, 0)


def _gather_add(ta, tb, src, dst):
    f = pl.kernel(
        _gather_add_body,
        out_type=jax.ShapeDtypeStruct((N_EDGES, DIM), jnp.float32),
        mesh=_sc_mesh(),
        scratch_types=[
            pltpu.VMEM((_KB,), jnp.int32),
            pltpu.VMEM((_KB,), jnp.int32),
            pltpu.VMEM((_KB, DIM), jnp.float32),
            pltpu.VMEM((_KB, DIM), jnp.float32),
            pltpu.SemaphoreType.DMA,
            pltpu.SemaphoreType.DMA,
        ],
    )
    return f(ta, tb, src, dst)


_HALF = DIM // 2
_NPAD = 10240
_RPT = _NPAD // _NS
_KZ = 128
_EPT = N_EDGES // _NS
_KS = 80


def _segsum_body(y_lo, y_hi, idx_hbm, d_lo, d_hi, zbuf, wbuf, ybuf, idx, acc):
    cid = lax.axis_index("c")
    tid = lax.axis_index("s")

    def zrow(i, c):
        for g in range(_HALF // _L):
            zbuf[i, pl.ds(g * _L, _L)] = jnp.zeros((_L,), jnp.float32)
        return c

    lax.fori_loop(0, _KZ, zrow, 0)
    for k in range(_RPT // _KZ):
        pltpu.sync_copy(zbuf, acc.at[pl.ds(tid * _RPT + k * _KZ, _KZ)])
    plsc.subcore_barrier()

    def scatter_loop(y_hbm):
        def chunk(j, c):
            base = tid * _EPT + j * _KS
            pltpu.sync_copy(idx_hbm.at[pl.ds(base, _KS)], idx)
            pltpu.sync_copy(y_hbm.at[pl.ds(base, _KS)], ybuf)
            pltpu.sync_copy(ybuf, acc.at[idx], add=True)
            return c

        lax.fori_loop(0, _EPT // _KS, chunk, 0)

    def writeout(d_out):
        for k in range(_RPT // _KZ):
            sl = pl.ds(tid * _RPT + k * _KZ, _KZ)
            pltpu.sync_copy(acc.at[sl], wbuf)
            pltpu.sync_copy(wbuf, d_out.at[sl])

    @pl.when(cid == 0)
    def _():
        scatter_loop(y_lo)

    @pl.when(cid == 1)
    def _():
        scatter_loop(y_hi)

    plsc.subcore_barrier()

    @pl.when(cid == 0)
    def _():
        writeout(d_lo)

    @pl.when(cid == 1)
    def _():
        writeout(d_hi)


def _segsum(y_lo, y_hi, idx):
    f = pl.kernel(
        _segsum_body,
        out_type=[jax.ShapeDtypeStruct((_NPAD, _HALF), jnp.float32)] * 2,
        mesh=_sc_mesh(),
        scratch_types=[
            pltpu.VMEM((_KZ, _HALF), jnp.float32),
            pltpu.VMEM((_KZ, _HALF), jnp.float32),
            pltpu.VMEM((_KS, _HALF), jnp.float32),
            pltpu.VMEM((_KS,), jnp.int32),
            pltpu.VMEM_SHARED((_NPAD, _HALF), jnp.float32),
        ],
    )
    return f(y_lo, y_hi, idx)


_BN_NODE = 1000
_BN_EDGE = 2000


def _node_feats_kernel(x_ref, w1, w2, w3, w4, b2_ref, b4_ref, o1, o2, o3, o4):
    x = x_ref[...]
    o1[...] = jnp.dot(x, w1[...], preferred_element_type=jnp.float32)
    o2[...] = jnp.dot(x, w2[...], preferred_element_type=jnp.float32) + b2_ref[...]
    o3[...] = jnp.dot(x, w3[...], preferred_element_type=jnp.float32)
    o4[...] = jnp.dot(x, w4[...], preferred_element_type=jnp.float32) + b4_ref[...]


def _node_feats(x, w1t, w2t, w3t, w4t, b2, b4):
    nb = N_NODES // _BN_NODE
    return pl.pallas_call(
        _node_feats_kernel,
        grid=(nb,),
        in_specs=[pl.BlockSpec((_BN_NODE, DIM), lambda i: (i, 0))]
        + [pl.BlockSpec((DIM, DIM), lambda i: (0, 0))] * 4
        + [pl.BlockSpec((1, DIM), lambda i: (0, 0))] * 2,
        out_specs=[pl.BlockSpec((_BN_NODE, DIM), lambda i: (i, 0))] * 4,
        out_shape=[jax.ShapeDtypeStruct((N_NODES, DIM), jnp.float32)] * 4,
    )(x, w1t, w2t, w3t, w4t, b2, b4)


def _stats2_kernel(a_ref, b_ref, sa, qa, sb, qb):
    @pl.when(pl.program_id(0) == 0)
    def _():
        sa[...] = jnp.zeros_like(sa)
        qa[...] = jnp.zeros_like(qa)
        sb[...] = jnp.zeros_like(sb)
        qb[...] = jnp.zeros_like(qb)

    a = a_ref[...]
    b = b_ref[...]
    sa[...] += jnp.sum(a, axis=0, keepdims=True)
    qa[...] += jnp.sum(a * a, axis=0, keepdims=True)
    sb[...] += jnp.sum(b, axis=0, keepdims=True)
    qb[...] += jnp.sum(b * b, axis=0, keepdims=True)


def _stats2(a, b):
    nb = N_EDGES // _BN_EDGE
    return pl.pallas_call(
        _stats2_kernel,
        grid=(nb,),
        in_specs=[pl.BlockSpec((_BN_EDGE, DIM), lambda i: (i, 0))] * 2,
        out_specs=[pl.BlockSpec((1, DIM), lambda i: (0, 0))] * 4,
        out_shape=[jax.ShapeDtypeStruct((1, DIM), jnp.float32)] * 4,
    )(a, b)


def _bnmm_kernel(u_ref, s_ref, t_ref, wt_ref, b_ref, z_ref, sz, qz):
    @pl.when(pl.program_id(0) == 0)
    def _():
        sz[...] = jnp.zeros_like(sz)
        qz[...] = jnp.zeros_like(qz)

    h = jnp.maximum(u_ref[...] * s_ref[...] + t_ref[...], 0.0)
    z = jnp.dot(h, wt_ref[...], preferred_element_type=jnp.float32) + b_ref[...]
    z_ref[...] = z
    sz[...] += jnp.sum(z, axis=0, keepdims=True)
    qz[...] += jnp.sum(z * z, axis=0, keepdims=True)


def _bnmm(u, s, t, wt, b):
    nb = N_EDGES // _BN_EDGE
    return pl.pallas_call(
        _bnmm_kernel,
        grid=(nb,),
        in_specs=[pl.BlockSpec((_BN_EDGE, DIM), lambda i: (i, 0)),
                  pl.BlockSpec((1, DIM), lambda i: (0, 0)),
                  pl.BlockSpec((1, DIM), lambda i: (0, 0)),
                  pl.BlockSpec((DIM, DIM), lambda i: (0, 0)),
                  pl.BlockSpec((1, DIM), lambda i: (0, 0))],
        out_specs=[pl.BlockSpec((_BN_EDGE, DIM), lambda i: (i, 0)),
                   pl.BlockSpec((1, DIM), lambda i: (0, 0)),
                   pl.BlockSpec((1, DIM), lambda i: (0, 0))],
        out_shape=[jax.ShapeDtypeStruct((N_EDGES, DIM), jnp.float32),
                   jax.ShapeDtypeStruct((1, DIM), jnp.float32),
                   jax.ShapeDtypeStruct((1, DIM), jnp.float32)],
    )(u, s, t, wt, b)


def _bnrelu_split_kernel(z_ref, s_ref, t_ref, lo_ref, hi_ref):
    y = jnp.maximum(z_ref[...] * s_ref[...] + t_ref[...], 0.0)
    lo_ref[...] = y[:, :_HALF]
    hi_ref[...] = y[:, _HALF:]


def _bnrelu_split(z, s, t):
    nb = N_EDGES // _BN_EDGE
    return pl.pallas_call(
        _bnrelu_split_kernel,
        grid=(nb,),
        in_specs=[pl.BlockSpec((_BN_EDGE, DIM), lambda i: (i, 0)),
                  pl.BlockSpec((1, DIM), lambda i: (0, 0)),
                  pl.BlockSpec((1, DIM), lambda i: (0, 0))],
        out_specs=[pl.BlockSpec((_BN_EDGE, _HALF), lambda i: (i, 0))] * 2,
        out_shape=[jax.ShapeDtypeStruct((N_EDGES, _HALF), jnp.float32)] * 2,
    )(z, s, t)


def _final_kernel(x_ref, dfl, dfh, dol, doh, wt_ref, b_ref, g_ref, bb_ref,
                  out_ref):
    d = jnp.concatenate([dfl[...] + dol[...], dfh[...] + doh[...]], axis=1)
    xn = x_ref[...] + d
    y = jnp.dot(xn, wt_ref[...], preferred_element_type=jnp.float32)
    y = y + b_ref[...]
    m = jnp.mean(y, axis=0, keepdims=True)
    v = jnp.mean((y - m) ** 2, axis=0, keepdims=True)
    out_ref[...] = jnp.maximum(
        (y - m) / jnp.sqrt(v + EPS) * g_ref[...] + bb_ref[...], 0.0)


def _final_stage(x, d_parts, W, b, g, bb):
    dfl, dfh, dol, doh = d_parts
    return pl.pallas_call(
        _final_kernel,
        out_shape=jax.ShapeDtypeStruct((N_NODES, DIM), jnp.float32),
    )(x, dfl[:N_NODES], dfh[:N_NODES], dol[:N_NODES], doh[:N_NODES],
      W.T, b.reshape(1, DIM), g.reshape(1, DIM), bb.reshape(1, DIM))


def _affine(sum_, sumsq, g, b, n):
    m = sum_ / n
    v = sumsq / n - m * m
    s = g.reshape(1, DIM) / jnp.sqrt(v + EPS)
    t = b.reshape(1, DIM) - m * s
    return s, t


def kernel(x, edges, params):
    p = params
    src = jnp.asarray(edges[:, 0], jnp.int32)
    dst = jnp.asarray(edges[:, 1], jnp.int32)

    fia, fib, foa, fob = _node_feats(
        x, p['FI_fc1a_W'].T, p['FI_fc1b_W'].T, p['FO_fc1a_W'].T,
        p['FO_fc1b_W'].T, p['FI_fc1b_b'].reshape(1, DIM),
        p['FO_fc1b_b'].reshape(1, DIM))

    fi_in = _gather_add(fia, fib, src, dst)
    fo_in = _gather_add(foa, fob, src, dst)

    s_fi, q_fi, s_fo, q_fo = _stats2(fi_in, fo_in)
    s1i, t1i = _affine(s_fi, q_fi, p['FI_bn1_g'], p['FI_bn1_b'], N_EDGES)
    s1o, t1o = _affine(s_fo, q_fo, p['FO_bn1_g'], p['FO_bn1_b'], N_EDGES)

    z_fi, sz_fi, qz_fi = _bnmm(fi_in, s1i, t1i, p['FI_fc2_W'].T,
                               p['FI_fc2_b'].reshape(1, DIM))
    s2i, t2i = _affine(sz_fi, qz_fi, p['FI_bn2_g'], p['FI_bn2_b'], N_EDGES)
    yfi_lo, yfi_hi = _bnrelu_split(z_fi, s2i, t2i)
    dfi_lo, dfi_hi = _segsum(yfi_lo, yfi_hi, dst)

    z_fo, sz_fo, qz_fo = _bnmm(fo_in, s1o, t1o, p['FO_fc2_W'].T,
                               p['FO_fc2_b'].reshape(1, DIM))
    s2o, t2o = _affine(sz_fo, qz_fo, p['FO_bn2_g'], p['FO_bn2_b'], N_EDGES)
    yfo_lo, yfo_hi = _bnrelu_split(z_fo, s2o, t2o)
    dfo_lo, dfo_hi = _segsum(yfo_lo, yfo_hi, src)

    return _final_stage(x, (dfi_lo, dfi_hi, dfo_lo, dfo_hi),
                        p['FP_fc_W'], p['FP_fc_b'], p['FP_bn_g'], p['FP_bn_b'])


# speedup vs baseline: 2.4029x; 1.0754x over previous
import functools

import jax
import jax.numpy as jnp
from jax import lax
from jax.experimental import pallas as pl
from jax.experimental.pallas import tpu as pltpu
from jax.experimental.pallas import tpu_sc as plsc

DIM = 256
N_NODES = 10000
N_EDGES = 160000
EPS = 1e-5

_SC_INFO = plsc.get_sparse_core_info()
_NC, _NS, _L = _SC_INFO.num_cores, _SC_INFO.num_subcores, _SC_INFO.num_lanes
_NW = _NC * _NS

_sc_mesh = functools.partial(
    plsc.VectorSubcoreMesh, core_axis_name="c", subcore_axis_name="s")


_EPW = N_EDGES // _NW
_KB = 200
_NCHUNK = _EPW // _KB


def _gather_add_body(ta, tb, src_hbm, dst_hbm, out, stats_out,
                     idx_s, idx_d, bufa, bufb, statbuf, sem_a, sem_b):
    wid = lax.axis_index("s") * _NC + lax.axis_index("c")
    base0 = wid * _EPW
    ng = DIM // _L
    zero = jnp.zeros((_L,), jnp.float32)

    def chunk(j, carry):
        base = base0 + j * _KB
        pltpu.sync_copy(src_hbm.at[pl.ds(base, _KB)], idx_s)
        pltpu.sync_copy(dst_hbm.at[pl.ds(base, _KB)], idx_d)
        cp_a = pltpu.async_copy(ta.at[idx_s], bufa, sem_a)
        cp_b = pltpu.async_copy(tb.at[idx_d], bufb, sem_b)
        cp_a.wait()
        cp_b.wait()

        def row(i, c):
            acc = list(c)
            for g in range(ng):
                sl = pl.ds(g * _L, _L)
                v = bufa[i, sl] + bufb[i, sl]
                bufa[i, sl] = v
                acc[g] = acc[g] + v
                acc[ng + g] = acc[ng + g] + v * v
            return tuple(acc)

        carry = lax.fori_loop(0, _KB, row, carry, unroll=2)
        pltpu.sync_copy(bufa, out.at[pl.ds(base, _KB)])
        return carry

    init = (zero,) * (2 * ng)
    fin = lax.fori_loop(0, _NCHUNK, chunk, init)
    for g in range(ng):
        statbuf[pl.ds(g * _L, _L)] = fin[g]
        statbuf[pl.ds(DIM + g * _L, _L)] = fin[ng + g]
    pltpu.sync_copy(statbuf, stats_out.at[wid])


def _gather_add(ta, tb, src, dst):
    f = pl.kernel(
        _gather_add_body,
        out_type=[jax.ShapeDtypeStruct((N_EDGES, DIM), jnp.float32),
                  jax.ShapeDtypeStruct((_NW, 2 * DIM), jnp.float32)],
        mesh=_sc_mesh(),
        scratch_types=[
            pltpu.VMEM((_KB,), jnp.int32),
            pltpu.VMEM((_KB,), jnp.int32),
            pltpu.VMEM((_KB, DIM), jnp.float32),
            pltpu.VMEM((_KB, DIM), jnp.float32),
            pltpu.VMEM((2 * DIM,), jnp.float32),
            pltpu.SemaphoreType.DMA,
            pltpu.SemaphoreType.DMA,
        ],
    )
    u, st = f(ta, tb, src, dst)
    sums = jnp.sum(st, axis=0)
    return u, sums[:DIM].reshape(1, DIM), sums[DIM:].reshape(1, DIM)


_HALF = DIM // 2
_NPAD = 10240
_RPT = _NPAD // _NS
_KZ = 128
_EPT = N_EDGES // _NS
_KS = 80


def _segsum_body(y_lo, y_hi, idx_hbm, d_lo, d_hi, zbuf, wbuf, ybuf, idx, acc):
    cid = lax.axis_index("c")
    tid = lax.axis_index("s")

    def zrow(i, c):
        for g in range(_HALF // _L):
            zbuf[i, pl.ds(g * _L, _L)] = jnp.zeros((_L,), jnp.float32)
        return c

    lax.fori_loop(0, _KZ, zrow, 0)
    for k in range(_RPT // _KZ):
        pltpu.sync_copy(zbuf, acc.at[pl.ds(tid * _RPT + k * _KZ, _KZ)])
    plsc.subcore_barrier()

    def scatter_loop(y_hbm):
        def chunk(j, c):
            base = tid * _EPT + j * _KS
            pltpu.sync_copy(idx_hbm.at[pl.ds(base, _KS)], idx)
            pltpu.sync_copy(y_hbm.at[pl.ds(base, _KS)], ybuf)
            pltpu.sync_copy(ybuf, acc.at[idx], add=True)
            return c

        lax.fori_loop(0, _EPT // _KS, chunk, 0)

    def writeout(d_out):
        for k in range(_RPT // _KZ):
            sl = pl.ds(tid * _RPT + k * _KZ, _KZ)
            pltpu.sync_copy(acc.at[sl], wbuf)
            pltpu.sync_copy(wbuf, d_out.at[sl])

    @pl.when(cid == 0)
    def _():
        scatter_loop(y_lo)

    @pl.when(cid == 1)
    def _():
        scatter_loop(y_hi)

    plsc.subcore_barrier()

    @pl.when(cid == 0)
    def _():
        writeout(d_lo)

    @pl.when(cid == 1)
    def _():
        writeout(d_hi)


def _segsum(y_lo, y_hi, idx):
    f = pl.kernel(
        _segsum_body,
        out_type=[jax.ShapeDtypeStruct((_NPAD, _HALF), jnp.float32)] * 2,
        mesh=_sc_mesh(),
        scratch_types=[
            pltpu.VMEM((_KZ, _HALF), jnp.float32),
            pltpu.VMEM((_KZ, _HALF), jnp.float32),
            pltpu.VMEM((_KS, _HALF), jnp.float32),
            pltpu.VMEM((_KS,), jnp.int32),
            pltpu.VMEM_SHARED((_NPAD, _HALF), jnp.float32),
        ],
    )
    return f(y_lo, y_hi, idx)


_BN_NODE = 1000
_BN_EDGE = 2000


def _node_feats_kernel(x_ref, w1, w2, w3, w4, b2_ref, b4_ref, o1, o2, o3, o4):
    x = x_ref[...]
    o1[...] = jnp.dot(x, w1[...], preferred_element_type=jnp.float32)
    o2[...] = jnp.dot(x, w2[...], preferred_element_type=jnp.float32) + b2_ref[...]
    o3[...] = jnp.dot(x, w3[...], preferred_element_type=jnp.float32)
    o4[...] = jnp.dot(x, w4[...], preferred_element_type=jnp.float32) + b4_ref[...]


def _node_feats(x, w1t, w2t, w3t, w4t, b2, b4):
    nb = N_NODES // _BN_NODE
    return pl.pallas_call(
        _node_feats_kernel,
        grid=(nb,),
        in_specs=[pl.BlockSpec((_BN_NODE, DIM), lambda i: (i, 0))]
        + [pl.BlockSpec((DIM, DIM), lambda i: (0, 0))] * 4
        + [pl.BlockSpec((1, DIM), lambda i: (0, 0))] * 2,
        out_specs=[pl.BlockSpec((_BN_NODE, DIM), lambda i: (i, 0))] * 4,
        out_shape=[jax.ShapeDtypeStruct((N_NODES, DIM), jnp.float32)] * 4,
    )(x, w1t, w2t, w3t, w4t, b2, b4)


def _bnmm_kernel(u_ref, s_ref, t_ref, wt_ref, b_ref, z_ref, sz, qz):
    @pl.when(pl.program_id(0) == 0)
    def _():
        sz[...] = jnp.zeros_like(sz)
        qz[...] = jnp.zeros_like(qz)

    h = jnp.maximum(u_ref[...] * s_ref[...] + t_ref[...], 0.0)
    z = jnp.dot(h, wt_ref[...], preferred_element_type=jnp.float32) + b_ref[...]
    z_ref[...] = z
    sz[...] += jnp.sum(z, axis=0, keepdims=True)
    qz[...] += jnp.sum(z * z, axis=0, keepdims=True)


def _bnmm(u, s, t, wt, b):
    nb = N_EDGES // _BN_EDGE
    return pl.pallas_call(
        _bnmm_kernel,
        grid=(nb,),
        in_specs=[pl.BlockSpec((_BN_EDGE, DIM), lambda i: (i, 0)),
                  pl.BlockSpec((1, DIM), lambda i: (0, 0)),
                  pl.BlockSpec((1, DIM), lambda i: (0, 0)),
                  pl.BlockSpec((DIM, DIM), lambda i: (0, 0)),
                  pl.BlockSpec((1, DIM), lambda i: (0, 0))],
        out_specs=[pl.BlockSpec((_BN_EDGE, DIM), lambda i: (i, 0)),
                   pl.BlockSpec((1, DIM), lambda i: (0, 0)),
                   pl.BlockSpec((1, DIM), lambda i: (0, 0))],
        out_shape=[jax.ShapeDtypeStruct((N_EDGES, DIM), jnp.float32),
                   jax.ShapeDtypeStruct((1, DIM), jnp.float32),
                   jax.ShapeDtypeStruct((1, DIM), jnp.float32)],
    )(u, s, t, wt, b)


def _bnrelu_split_kernel(z_ref, s_ref, t_ref, lo_ref, hi_ref):
    y = jnp.maximum(z_ref[...] * s_ref[...] + t_ref[...], 0.0)
    lo_ref[...] = y[:, :_HALF]
    hi_ref[...] = y[:, _HALF:]


def _bnrelu_split(z, s, t):
    nb = N_EDGES // _BN_EDGE
    return pl.pallas_call(
        _bnrelu_split_kernel,
        grid=(nb,),
        in_specs=[pl.BlockSpec((_BN_EDGE, DIM), lambda i: (i, 0)),
                  pl.BlockSpec((1, DIM), lambda i: (0, 0)),
                  pl.BlockSpec((1, DIM), lambda i: (0, 0))],
        out_specs=[pl.BlockSpec((_BN_EDGE, _HALF), lambda i: (i, 0))] * 2,
        out_shape=[jax.ShapeDtypeStruct((N_EDGES, _HALF), jnp.float32)] * 2,
    )(z, s, t)


def _final_kernel(x_ref, dfl, dfh, dol, doh, wt_ref, b_ref, g_ref, bb_ref,
                  out_ref):
    d = jnp.concatenate([dfl[...] + dol[...], dfh[...] + doh[...]], axis=1)
    xn = x_ref[...] + d
    y = jnp.dot(xn, wt_ref[...], preferred_element_type=jnp.float32)
    y = y + b_ref[...]
    m = jnp.mean(y, axis=0, keepdims=True)
    v = jnp.mean((y - m) ** 2, axis=0, keepdims=True)
    out_ref[...] = jnp.maximum(
        (y - m) / jnp.sqrt(v + EPS) * g_ref[...] + bb_ref[...], 0.0)


def _final_stage(x, d_parts, W, b, g, bb):
    dfl, dfh, dol, doh = d_parts
    return pl.pallas_call(
        _final_kernel,
        out_shape=jax.ShapeDtypeStruct((N_NODES, DIM), jnp.float32),
    )(x, dfl[:N_NODES], dfh[:N_NODES], dol[:N_NODES], doh[:N_NODES],
      W.T, b.reshape(1, DIM), g.reshape(1, DIM), bb.reshape(1, DIM))


def _affine(sum_, sumsq, g, b, n):
    m = sum_ / n
    v = sumsq / n - m * m
    s = g.reshape(1, DIM) / jnp.sqrt(v + EPS)
    t = b.reshape(1, DIM) - m * s
    return s, t


def kernel(x, edges, params):
    p = params
    src = jnp.asarray(edges[:, 0], jnp.int32)
    dst = jnp.asarray(edges[:, 1], jnp.int32)

    fia, fib, foa, fob = _node_feats(
        x, p['FI_fc1a_W'].T, p['FI_fc1b_W'].T, p['FO_fc1a_W'].T,
        p['FO_fc1b_W'].T, p['FI_fc1b_b'].reshape(1, DIM),
        p['FO_fc1b_b'].reshape(1, DIM))

    fi_in, s_fi, q_fi = _gather_add(fia, fib, src, dst)
    fo_in, s_fo, q_fo = _gather_add(foa, fob, src, dst)

    s1i, t1i = _affine(s_fi, q_fi, p['FI_bn1_g'], p['FI_bn1_b'], N_EDGES)
    s1o, t1o = _affine(s_fo, q_fo, p['FO_bn1_g'], p['FO_bn1_b'], N_EDGES)

    z_fi, sz_fi, qz_fi = _bnmm(fi_in, s1i, t1i, p['FI_fc2_W'].T,
                               p['FI_fc2_b'].reshape(1, DIM))
    s2i, t2i = _affine(sz_fi, qz_fi, p['FI_bn2_g'], p['FI_bn2_b'], N_EDGES)
    yfi_lo, yfi_hi = _bnrelu_split(z_fi, s2i, t2i)
    dfi_lo, dfi_hi = _segsum(yfi_lo, yfi_hi, dst)

    z_fo, sz_fo, qz_fo = _bnmm(fo_in, s1o, t1o, p['FO_fc2_W'].T,
                               p['FO_fc2_b'].reshape(1, DIM))
    s2o, t2o = _affine(sz_fo, qz_fo, p['FO_bn2_g'], p['FO_bn2_b'], N_EDGES)
    yfo_lo, yfo_hi = _bnrelu_split(z_fo, s2o, t2o)
    dfo_lo, dfo_hi = _segsum(yfo_lo, yfo_hi, src)

    return _final_stage(x, (dfi_lo, dfi_hi, dfo_lo, dfo_hi),
                        p['FP_fc_W'], p['FP_fc_b'], p['FP_bn_g'], p['FP_bn_b'])


# speedup vs baseline: 3.1422x; 1.3077x over previous
import functools

import jax
import jax.numpy as jnp
from jax import lax
from jax.experimental import pallas as pl
from jax.experimental.pallas import tpu as pltpu
from jax.experimental.pallas import tpu_sc as plsc

DIM = 256
N_NODES = 10000
N_EDGES = 160000
EPS = 1e-5

_SC_INFO = plsc.get_sparse_core_info()
_NC, _NS, _L = _SC_INFO.num_cores, _SC_INFO.num_subcores, _SC_INFO.num_lanes
_NW = _NC * _NS

_sc_mesh = functools.partial(
    plsc.VectorSubcoreMesh, core_axis_name="c", subcore_axis_name="s")


_EPW = N_EDGES // _NW
_KB = 40
_NCHUNK = _EPW // _KB
_NPAIR = (_NCHUNK - 1) // 2


def _gather_add_body(ta, tb, src_hbm, dst_hbm, out, stats_out,
                     idx_s, idx_d, bufa0, bufb0, bufa1, bufb1, statbuf,
                     sem_a0, sem_b0, sem_a1, sem_b1, sem_w0, sem_w1):
    wid = lax.axis_index("s") * _NC + lax.axis_index("c")
    base0 = wid * _EPW
    ng = DIM // _L
    zero = jnp.zeros((_L,), jnp.float32)

    pltpu.sync_copy(src_hbm.at[pl.ds(base0, _EPW)], idx_s)
    pltpu.sync_copy(dst_hbm.at[pl.ds(base0, _EPW)], idx_d)

    def issue(j, bufa, bufb, sa, sb):
        off = j * _KB
        pltpu.async_copy(ta.at[idx_s.at[pl.ds(off, _KB)]], bufa, sa)
        pltpu.async_copy(tb.at[idx_d.at[pl.ds(off, _KB)]], bufb, sb)

    def wait_gather(j, bufa, bufb, sa, sb):
        off = j * _KB
        pltpu.make_async_copy(ta.at[idx_s.at[pl.ds(off, _KB)]], bufa, sa).wait()
        pltpu.make_async_copy(tb.at[idx_d.at[pl.ds(off, _KB)]], bufb, sb).wait()

    def compute(bufa, bufb, carry):
        def row(i, c):
            acc = list(c)
            for g in range(ng):
                sl = pl.ds(g * _L, _L)
                v = bufa[i, sl] + bufb[i, sl]
                bufa[i, sl] = v
                acc[g] = acc[g] + v
                acc[ng + g] = acc[ng + g] + v * v
            return tuple(acc)

        return lax.fori_loop(0, _KB, row, carry, unroll=2)

    issue(0, bufa0, bufb0, sem_a0, sem_b0)
    issue(1, bufa1, bufb1, sem_a1, sem_b1)

    def pair(k, carry):
        c0 = 2 * k
        wait_gather(c0, bufa0, bufb0, sem_a0, sem_b0)
        carry = compute(bufa0, bufb0, carry)
        pltpu.async_copy(bufa0, out.at[pl.ds(base0 + c0 * _KB, _KB)], sem_w0)
        wait_gather(c0 + 1, bufa1, bufb1, sem_a1, sem_b1)
        carry = compute(bufa1, bufb1, carry)
        pltpu.async_copy(bufa1, out.at[pl.ds(base0 + (c0 + 1) * _KB, _KB)],
                         sem_w1)
        pltpu.make_async_copy(
            bufa0, out.at[pl.ds(base0 + c0 * _KB, _KB)], sem_w0).wait()
        issue(c0 + 2, bufa0, bufb0, sem_a0, sem_b0)
        pltpu.make_async_copy(
            bufa1, out.at[pl.ds(base0 + (c0 + 1) * _KB, _KB)], sem_w1).wait()

        @pl.when(c0 + 3 < _NCHUNK)
        def _():
            issue(c0 + 3, bufa1, bufb1, sem_a1, sem_b1)

        return carry

    init = (zero,) * (2 * ng)
    fin = lax.fori_loop(0, _NPAIR, pair, init)

    last = _NCHUNK - 1
    wait_gather(last, bufa0, bufb0, sem_a0, sem_b0)
    fin = compute(bufa0, bufb0, fin)
    pltpu.sync_copy(bufa0, out.at[pl.ds(base0 + last * _KB, _KB)])

    for g in range(ng):
        statbuf[pl.ds(g * _L, _L)] = fin[g]
        statbuf[pl.ds(DIM + g * _L, _L)] = fin[ng + g]
    pltpu.sync_copy(statbuf, stats_out.at[wid])


def _gather_add(ta, tb, src, dst):
    f = pl.kernel(
        _gather_add_body,
        out_type=[jax.ShapeDtypeStruct((N_EDGES, DIM), jnp.float32),
                  jax.ShapeDtypeStruct((_NW, 2 * DIM), jnp.float32)],
        mesh=_sc_mesh(),
        scratch_types=[
            pltpu.VMEM((_EPW,), jnp.int32),
            pltpu.VMEM((_EPW,), jnp.int32),
            pltpu.VMEM((_KB, DIM), jnp.float32),
            pltpu.VMEM((_KB, DIM), jnp.float32),
            pltpu.VMEM((_KB, DIM), jnp.float32),
            pltpu.VMEM((_KB, DIM), jnp.float32),
            pltpu.VMEM((2 * DIM,), jnp.float32),
            pltpu.SemaphoreType.DMA,
            pltpu.SemaphoreType.DMA,
            pltpu.SemaphoreType.DMA,
            pltpu.SemaphoreType.DMA,
            pltpu.SemaphoreType.DMA,
            pltpu.SemaphoreType.DMA,
        ],
    )
    u, st = f(ta, tb, src, dst)
    sums = jnp.sum(st, axis=0)
    return u, sums[:DIM].reshape(1, DIM), sums[DIM:].reshape(1, DIM)


_HALF = DIM // 2
_NPAD = 10240
_RPT = _NPAD // _NS
_KZ = 128
_EPT = N_EDGES // _NS
_KS = 80


def _segsum_body(y_lo, y_hi, idx_hbm, d_lo, d_hi, wbuf,
                 ybuf0, ybuf1, idx0, idx1, acc,
                 sem_i0, sem_y0, sem_i1, sem_y1):
    cid = lax.axis_index("c")
    tid = lax.axis_index("s")
    nch = _EPT // _KS
    npair = (nch - 1) // 2

    def zrow(i, c):
        for g in range(_HALF // _L):
            wbuf[i, pl.ds(g * _L, _L)] = jnp.zeros((_L,), jnp.float32)
        return c

    lax.fori_loop(0, _KS, zrow, 0)
    for k in range(_RPT // _KS):
        pltpu.sync_copy(wbuf, acc.at[pl.ds(tid * _RPT + k * _KS, _KS)])
    plsc.subcore_barrier()

    def scatter_loop(y_hbm):
        def issue(j, ib, yb, si, sy):
            base = tid * _EPT + j * _KS
            pltpu.async_copy(idx_hbm.at[pl.ds(base, _KS)], ib, si)
            pltpu.async_copy(y_hbm.at[pl.ds(base, _KS)], yb, sy)

        def wait_load(j, ib, yb, si, sy):
            base = tid * _EPT + j * _KS
            pltpu.make_async_copy(idx_hbm.at[pl.ds(base, _KS)], ib, si).wait()
            pltpu.make_async_copy(y_hbm.at[pl.ds(base, _KS)], yb, sy).wait()

        issue(0, idx0, ybuf0, sem_i0, sem_y0)
        issue(1, idx1, ybuf1, sem_i1, sem_y1)

        def pair(k, c):
            c0 = 2 * k
            wait_load(c0, idx0, ybuf0, sem_i0, sem_y0)
            pltpu.sync_copy(ybuf0, acc.at[idx0], add=True)
            issue(c0 + 2, idx0, ybuf0, sem_i0, sem_y0)
            wait_load(c0 + 1, idx1, ybuf1, sem_i1, sem_y1)
            pltpu.sync_copy(ybuf1, acc.at[idx1], add=True)

            @pl.when(c0 + 3 < nch)
            def _():
                issue(c0 + 3, idx1, ybuf1, sem_i1, sem_y1)

            return c

        lax.fori_loop(0, npair, pair, 0)
        wait_load(nch - 1, idx0, ybuf0, sem_i0, sem_y0)
        pltpu.sync_copy(ybuf0, acc.at[idx0], add=True)

    def writeout(d_out):
        for k in range(_RPT // _KS):
            sl = pl.ds(tid * _RPT + k * _KS, _KS)
            pltpu.sync_copy(acc.at[sl], wbuf)
            pltpu.sync_copy(wbuf, d_out.at[sl])

    @pl.when(cid == 0)
    def _():
        scatter_loop(y_lo)

    @pl.when(cid == 1)
    def _():
        scatter_loop(y_hi)

    plsc.subcore_barrier()

    @pl.when(cid == 0)
    def _():
        writeout(d_lo)

    @pl.when(cid == 1)
    def _():
        writeout(d_hi)


def _segsum(y_lo, y_hi, idx):
    f = pl.kernel(
        _segsum_body,
        out_type=[jax.ShapeDtypeStruct((_NPAD, _HALF), jnp.float32)] * 2,
        mesh=_sc_mesh(),
        scratch_types=[
            pltpu.VMEM((_KS, _HALF), jnp.float32),
            pltpu.VMEM((_KS, _HALF), jnp.float32),
            pltpu.VMEM((_KS, _HALF), jnp.float32),
            pltpu.VMEM((_KS,), jnp.int32),
            pltpu.VMEM((_KS,), jnp.int32),
            pltpu.VMEM_SHARED((_NPAD, _HALF), jnp.float32),
            pltpu.SemaphoreType.DMA,
            pltpu.SemaphoreType.DMA,
            pltpu.SemaphoreType.DMA,
            pltpu.SemaphoreType.DMA,
        ],
    )
    return f(y_lo, y_hi, idx)


_BN_NODE = 1000
_BN_EDGE = 2000


def _node_feats_kernel(x_ref, w1, w2, w3, w4, b2_ref, b4_ref, o1, o2, o3, o4):
    x = x_ref[...]
    o1[...] = jnp.dot(x, w1[...], preferred_element_type=jnp.float32)
    o2[...] = jnp.dot(x, w2[...], preferred_element_type=jnp.float32) + b2_ref[...]
    o3[...] = jnp.dot(x, w3[...], preferred_element_type=jnp.float32)
    o4[...] = jnp.dot(x, w4[...], preferred_element_type=jnp.float32) + b4_ref[...]


def _node_feats(x, w1t, w2t, w3t, w4t, b2, b4):
    nb = N_NODES // _BN_NODE
    return pl.pallas_call(
        _node_feats_kernel,
        grid=(nb,),
        in_specs=[pl.BlockSpec((_BN_NODE, DIM), lambda i: (i, 0))]
        + [pl.BlockSpec((DIM, DIM), lambda i: (0, 0))] * 4
        + [pl.BlockSpec((1, DIM), lambda i: (0, 0))] * 2,
        out_specs=[pl.BlockSpec((_BN_NODE, DIM), lambda i: (i, 0))] * 4,
        out_shape=[jax.ShapeDtypeStruct((N_NODES, DIM), jnp.float32)] * 4,
    )(x, w1t, w2t, w3t, w4t, b2, b4)


def _bnmm_kernel(u_ref, s_ref, t_ref, wt_ref, b_ref, z_ref, sz, qz):
    @pl.when(pl.program_id(0) == 0)
    def _():
        sz[...] = jnp.zeros_like(sz)
        qz[...] = jnp.zeros_like(qz)

    h = jnp.maximum(u_ref[...] * s_ref[...] + t_ref[...], 0.0)
    z = jnp.dot(h, wt_ref[...], preferred_element_type=jnp.float32) + b_ref[...]
    z_ref[...] = z
    sz[...] += jnp.sum(z, axis=0, keepdims=True)
    qz[...] += jnp.sum(z * z, axis=0, keepdims=True)


def _bnmm(u, s, t, wt, b):
    nb = N_EDGES // _BN_EDGE
    return pl.pallas_call(
        _bnmm_kernel,
        grid=(nb,),
        in_specs=[pl.BlockSpec((_BN_EDGE, DIM), lambda i: (i, 0)),
                  pl.BlockSpec((1, DIM), lambda i: (0, 0)),
                  pl.BlockSpec((1, DIM), lambda i: (0, 0)),
                  pl.BlockSpec((DIM, DIM), lambda i: (0, 0)),
                  pl.BlockSpec((1, DIM), lambda i: (0, 0))],
        out_specs=[pl.BlockSpec((_BN_EDGE, DIM), lambda i: (i, 0)),
                   pl.BlockSpec((1, DIM), lambda i: (0, 0)),
                   pl.BlockSpec((1, DIM), lambda i: (0, 0))],
        out_shape=[jax.ShapeDtypeStruct((N_EDGES, DIM), jnp.float32),
                   jax.ShapeDtypeStruct((1, DIM), jnp.float32),
                   jax.ShapeDtypeStruct((1, DIM), jnp.float32)],
    )(u, s, t, wt, b)


def _bnrelu_split_kernel(z_ref, s_ref, t_ref, lo_ref, hi_ref):
    y = jnp.maximum(z_ref[...] * s_ref[...] + t_ref[...], 0.0)
    lo_ref[...] = y[:, :_HALF]
    hi_ref[...] = y[:, _HALF:]


def _bnrelu_split(z, s, t):
    nb = N_EDGES // _BN_EDGE
    return pl.pallas_call(
        _bnrelu_split_kernel,
        grid=(nb,),
        in_specs=[pl.BlockSpec((_BN_EDGE, DIM), lambda i: (i, 0)),
                  pl.BlockSpec((1, DIM), lambda i: (0, 0)),
                  pl.BlockSpec((1, DIM), lambda i: (0, 0))],
        out_specs=[pl.BlockSpec((_BN_EDGE, _HALF), lambda i: (i, 0))] * 2,
        out_shape=[jax.ShapeDtypeStruct((N_EDGES, _HALF), jnp.float32)] * 2,
    )(z, s, t)


def _final_kernel(x_ref, dfl, dfh, dol, doh, wt_ref, b_ref, g_ref, bb_ref,
                  out_ref):
    d = jnp.concatenate([dfl[...] + dol[...], dfh[...] + doh[...]], axis=1)
    xn = x_ref[...] + d
    y = jnp.dot(xn, wt_ref[...], preferred_element_type=jnp.float32)
    y = y + b_ref[...]
    m = jnp.mean(y, axis=0, keepdims=True)
    v = jnp.mean((y - m) ** 2, axis=0, keepdims=True)
    out_ref[...] = jnp.maximum(
        (y - m) / jnp.sqrt(v + EPS) * g_ref[...] + bb_ref[...], 0.0)


def _final_stage(x, d_parts, W, b, g, bb):
    dfl, dfh, dol, doh = d_parts
    return pl.pallas_call(
        _final_kernel,
        out_shape=jax.ShapeDtypeStruct((N_NODES, DIM), jnp.float32),
    )(x, dfl[:N_NODES], dfh[:N_NODES], dol[:N_NODES], doh[:N_NODES],
      W.T, b.reshape(1, DIM), g.reshape(1, DIM), bb.reshape(1, DIM))


def _affine(sum_, sumsq, g, b, n):
    m = sum_ / n
    v = sumsq / n - m * m
    s = g.reshape(1, DIM) / jnp.sqrt(v + EPS)
    t = b.reshape(1, DIM) - m * s
    return s, t


def kernel(x, edges, params):
    p = params
    src = jnp.asarray(edges[:, 0], jnp.int32)
    dst = jnp.asarray(edges[:, 1], jnp.int32)

    fia, fib, foa, fob = _node_feats(
        x, p['FI_fc1a_W'].T, p['FI_fc1b_W'].T, p['FO_fc1a_W'].T,
        p['FO_fc1b_W'].T, p['FI_fc1b_b'].reshape(1, DIM),
        p['FO_fc1b_b'].reshape(1, DIM))

    fi_in, s_fi, q_fi = _gather_add(fia, fib, src, dst)
    fo_in, s_fo, q_fo = _gather_add(foa, fob, src, dst)

    s1i, t1i = _affine(s_fi, q_fi, p['FI_bn1_g'], p['FI_bn1_b'], N_EDGES)
    s1o, t1o = _affine(s_fo, q_fo, p['FO_bn1_g'], p['FO_bn1_b'], N_EDGES)

    z_fi, sz_fi, qz_fi = _bnmm(fi_in, s1i, t1i, p['FI_fc2_W'].T,
                               p['FI_fc2_b'].reshape(1, DIM))
    s2i, t2i = _affine(sz_fi, qz_fi, p['FI_bn2_g'], p['FI_bn2_b'], N_EDGES)
    yfi_lo, yfi_hi = _bnrelu_split(z_fi, s2i, t2i)
    dfi_lo, dfi_hi = _segsum(yfi_lo, yfi_hi, dst)

    z_fo, sz_fo, qz_fo = _bnmm(fo_in, s1o, t1o, p['FO_fc2_W'].T,
                               p['FO_fc2_b'].reshape(1, DIM))
    s2o, t2o = _affine(sz_fo, qz_fo, p['FO_bn2_g'], p['FO_bn2_b'], N_EDGES)
    yfo_lo, yfo_hi = _bnrelu_split(z_fo, s2o, t2o)
    dfo_lo, dfo_hi = _segsum(yfo_lo, yfo_hi, src)

    return _final_stage(x, (dfi_lo, dfi_hi, dfo_lo, dfo_hi),
                        p['FP_fc_W'], p['FP_fc_b'], p['FP_bn_g'], p['FP_bn_b'])


# speedup vs baseline: 3.4905x; 1.1108x over previous
import functools

import jax
import jax.numpy as jnp
from jax import lax
from jax.experimental import pallas as pl
from jax.experimental.pallas import tpu as pltpu
from jax.experimental.pallas import tpu_sc as plsc

DIM = 256
N_NODES = 10000
N_EDGES = 160000
EPS = 1e-5

_SC_INFO = plsc.get_sparse_core_info()
_NC, _NS, _L = _SC_INFO.num_cores, _SC_INFO.num_subcores, _SC_INFO.num_lanes
_NW = _NC * _NS

_sc_mesh = functools.partial(
    plsc.VectorSubcoreMesh, core_axis_name="c", subcore_axis_name="s")


_EPW = N_EDGES // _NW
_KB = 40
_NCHUNK = _EPW // _KB
_NPAIR = (_NCHUNK - 1) // 2


def _gather_add_body(ta, tb, src_hbm, dst_hbm, out, stats_out,
                     idx_s, idx_d, bufa0, bufb0, bufa1, bufb1,
                     bufa2, bufb2, bufa3, bufb3, statbuf,
                     sa0, sb0, sa1, sb1, sa2, sb2, sa3, sb3,
                     sw0, sw1, sw2, sw3):
    wid = lax.axis_index("s") * _NC + lax.axis_index("c")
    base0 = wid * _EPW
    ng = DIM // _L
    zero = jnp.zeros((_L,), jnp.float32)
    bufs = ((bufa0, bufb0, sa0, sb0, sw0), (bufa1, bufb1, sa1, sb1, sw1),
            (bufa2, bufb2, sa2, sb2, sw2), (bufa3, bufb3, sa3, sb3, sw3))

    pltpu.sync_copy(src_hbm.at[pl.ds(base0, _EPW)], idx_s)
    pltpu.sync_copy(dst_hbm.at[pl.ds(base0, _EPW)], idx_d)

    def issue(j, b):
        bufa, bufb, sa, sb, _ = bufs[b]
        off = j * _KB
        pltpu.async_copy(ta.at[idx_s.at[pl.ds(off, _KB)]], bufa, sa)
        pltpu.async_copy(tb.at[idx_d.at[pl.ds(off, _KB)]], bufb, sb)

    def wait_gather(j, b):
        bufa, bufb, sa, sb, _ = bufs[b]
        off = j * _KB
        pltpu.make_async_copy(ta.at[idx_s.at[pl.ds(off, _KB)]], bufa, sa).wait()
        pltpu.make_async_copy(tb.at[idx_d.at[pl.ds(off, _KB)]], bufb, sb).wait()

    def issue_w(j, b):
        bufa = bufs[b][0]
        pltpu.async_copy(bufa, out.at[pl.ds(base0 + j * _KB, _KB)], bufs[b][4])

    def wait_w(j, b):
        bufa = bufs[b][0]
        pltpu.make_async_copy(
            bufa, out.at[pl.ds(base0 + j * _KB, _KB)], bufs[b][4]).wait()

    def compute(b, carry):
        bufa, bufb = bufs[b][0], bufs[b][1]

        def row(i, c):
            acc = list(c)
            for g in range(ng):
                sl = pl.ds(g * _L, _L)
                v = bufa[i, sl] + bufb[i, sl]
                bufa[i, sl] = v
                acc[g] = acc[g] + v
                acc[ng + g] = acc[ng + g] + v * v
            return tuple(acc)

        return lax.fori_loop(0, _KB, row, carry, unroll=2)

    issue(0, 0)
    issue(1, 1)
    issue(2, 2)

    def quad(k, carry):
        c0 = 4 * k
        for b in range(4):
            c = c0 + b
            wait_gather(c, b)
            carry = compute(b, carry)
            issue_w(c, b)
            pb = (b - 1) % 4
            if b == 0:
                @pl.when(k > 0)
                def _():
                    wait_w(c - 1, pb)
                    issue(c + 3, pb)

                @pl.when(k == 0)
                def _():
                    issue(c + 3, pb)
            else:
                wait_w(c - 1, pb)

                @pl.when(c + 3 < _NCHUNK)
                def _():
                    issue(c + 3, pb)

        return carry

    init = (zero,) * (2 * ng)
    fin = lax.fori_loop(0, (_NCHUNK - 1) // 4, quad, init)

    last = _NCHUNK - 1
    wait_gather(last, 0)
    fin = compute(0, fin)
    issue_w(last, 0)
    wait_w(last - 1, 3)
    wait_w(last, 0)

    for g in range(ng):
        statbuf[pl.ds(g * _L, _L)] = fin[g]
        statbuf[pl.ds(DIM + g * _L, _L)] = fin[ng + g]
    pltpu.sync_copy(statbuf, stats_out.at[wid])


def _gather_add(ta, tb, src, dst):
    f = pl.kernel(
        _gather_add_body,
        out_type=[jax.ShapeDtypeStruct((N_EDGES, DIM), jnp.float32),
                  jax.ShapeDtypeStruct((_NW, 2 * DIM), jnp.float32)],
        mesh=_sc_mesh(),
        scratch_types=(
            [pltpu.VMEM((_EPW,), jnp.int32)] * 2
            + [pltpu.VMEM((_KB, DIM), jnp.float32)] * 8
            + [pltpu.VMEM((2 * DIM,), jnp.float32)]
            + [pltpu.SemaphoreType.DMA] * 12),
    )
    u, st = f(ta, tb, src, dst)
    sums = jnp.sum(st, axis=0)
    return u, sums[:DIM].reshape(1, DIM), sums[DIM:].reshape(1, DIM)


_HALF = DIM // 2
_NPAD = 10240
_RPT = _NPAD // _NS
_KZ = 128
_EPT = N_EDGES // _NS
_KS = 80


def _segsum_body(y_lo, y_hi, idx_hbm, d_lo, d_hi, wbuf,
                 ybuf0, ybuf1, idx0, idx1, acc,
                 sem_i0, sem_y0, sem_i1, sem_y1):
    cid = lax.axis_index("c")
    tid = lax.axis_index("s")
    nch = _EPT // _KS
    npair = (nch - 1) // 2

    def zrow(i, c):
        for g in range(_HALF // _L):
            wbuf[i, pl.ds(g * _L, _L)] = jnp.zeros((_L,), jnp.float32)
        return c

    lax.fori_loop(0, _KS, zrow, 0)
    for k in range(_RPT // _KS):
        pltpu.sync_copy(wbuf, acc.at[pl.ds(tid * _RPT + k * _KS, _KS)])
    plsc.subcore_barrier()

    def scatter_loop(y_hbm):
        def issue(j, ib, yb, si, sy):
            base = tid * _EPT + j * _KS
            pltpu.async_copy(idx_hbm.at[pl.ds(base, _KS)], ib, si)
            pltpu.async_copy(y_hbm.at[pl.ds(base, _KS)], yb, sy)

        def wait_load(j, ib, yb, si, sy):
            base = tid * _EPT + j * _KS
            pltpu.make_async_copy(idx_hbm.at[pl.ds(base, _KS)], ib, si).wait()
            pltpu.make_async_copy(y_hbm.at[pl.ds(base, _KS)], yb, sy).wait()

        issue(0, idx0, ybuf0, sem_i0, sem_y0)
        issue(1, idx1, ybuf1, sem_i1, sem_y1)

        def pair(k, c):
            c0 = 2 * k
            wait_load(c0, idx0, ybuf0, sem_i0, sem_y0)
            pltpu.sync_copy(ybuf0, acc.at[idx0], add=True)
            issue(c0 + 2, idx0, ybuf0, sem_i0, sem_y0)
            wait_load(c0 + 1, idx1, ybuf1, sem_i1, sem_y1)
            pltpu.sync_copy(ybuf1, acc.at[idx1], add=True)

            @pl.when(c0 + 3 < nch)
            def _():
                issue(c0 + 3, idx1, ybuf1, sem_i1, sem_y1)

            return c

        lax.fori_loop(0, npair, pair, 0)
        wait_load(nch - 1, idx0, ybuf0, sem_i0, sem_y0)
        pltpu.sync_copy(ybuf0, acc.at[idx0], add=True)

    def writeout(d_out):
        for k in range(_RPT // _KS):
            sl = pl.ds(tid * _RPT + k * _KS, _KS)
            pltpu.sync_copy(acc.at[sl], wbuf)
            pltpu.sync_copy(wbuf, d_out.at[sl])

    @pl.when(cid == 0)
    def _():
        scatter_loop(y_lo)

    @pl.when(cid == 1)
    def _():
        scatter_loop(y_hi)

    plsc.subcore_barrier()

    @pl.when(cid == 0)
    def _():
        writeout(d_lo)

    @pl.when(cid == 1)
    def _():
        writeout(d_hi)


def _segsum(y_lo, y_hi, idx):
    f = pl.kernel(
        _segsum_body,
        out_type=[jax.ShapeDtypeStruct((_NPAD, _HALF), jnp.float32)] * 2,
        mesh=_sc_mesh(),
        scratch_types=[
            pltpu.VMEM((_KS, _HALF), jnp.float32),
            pltpu.VMEM((_KS, _HALF), jnp.float32),
            pltpu.VMEM((_KS, _HALF), jnp.float32),
            pltpu.VMEM((_KS,), jnp.int32),
            pltpu.VMEM((_KS,), jnp.int32),
            pltpu.VMEM_SHARED((_NPAD, _HALF), jnp.float32),
            pltpu.SemaphoreType.DMA,
            pltpu.SemaphoreType.DMA,
            pltpu.SemaphoreType.DMA,
            pltpu.SemaphoreType.DMA,
        ],
    )
    return f(y_lo, y_hi, idx)


_BN_NODE = 1000
_BN_EDGE = 2000


def _node_feats_kernel(x_ref, w1, w2, w3, w4, b2_ref, b4_ref, o1, o2, o3, o4):
    x = x_ref[...]
    o1[...] = jnp.dot(x, w1[...], preferred_element_type=jnp.float32)
    o2[...] = jnp.dot(x, w2[...], preferred_element_type=jnp.float32) + b2_ref[...]
    o3[...] = jnp.dot(x, w3[...], preferred_element_type=jnp.float32)
    o4[...] = jnp.dot(x, w4[...], preferred_element_type=jnp.float32) + b4_ref[...]


def _node_feats(x, w1t, w2t, w3t, w4t, b2, b4):
    nb = N_NODES // _BN_NODE
    return pl.pallas_call(
        _node_feats_kernel,
        grid=(nb,),
        in_specs=[pl.BlockSpec((_BN_NODE, DIM), lambda i: (i, 0))]
        + [pl.BlockSpec((DIM, DIM), lambda i: (0, 0))] * 4
        + [pl.BlockSpec((1, DIM), lambda i: (0, 0))] * 2,
        out_specs=[pl.BlockSpec((_BN_NODE, DIM), lambda i: (i, 0))] * 4,
        out_shape=[jax.ShapeDtypeStruct((N_NODES, DIM), jnp.float32)] * 4,
    )(x, w1t, w2t, w3t, w4t, b2, b4)


def _bnmm_kernel(u_ref, s_ref, t_ref, wt_ref, b_ref, z_ref, sz, qz):
    @pl.when(pl.program_id(0) == 0)
    def _():
        sz[...] = jnp.zeros_like(sz)
        qz[...] = jnp.zeros_like(qz)

    h = jnp.maximum(u_ref[...] * s_ref[...] + t_ref[...], 0.0)
    z = jnp.dot(h, wt_ref[...], preferred_element_type=jnp.float32) + b_ref[...]
    z_ref[...] = z
    sz[...] += jnp.sum(z, axis=0, keepdims=True)
    qz[...] += jnp.sum(z * z, axis=0, keepdims=True)


def _bnmm(u, s, t, wt, b):
    nb = N_EDGES // _BN_EDGE
    return pl.pallas_call(
        _bnmm_kernel,
        grid=(nb,),
        in_specs=[pl.BlockSpec((_BN_EDGE, DIM), lambda i: (i, 0)),
                  pl.BlockSpec((1, DIM), lambda i: (0, 0)),
                  pl.BlockSpec((1, DIM), lambda i: (0, 0)),
                  pl.BlockSpec((DIM, DIM), lambda i: (0, 0)),
                  pl.BlockSpec((1, DIM), lambda i: (0, 0))],
        out_specs=[pl.BlockSpec((_BN_EDGE, DIM), lambda i: (i, 0)),
                   pl.BlockSpec((1, DIM), lambda i: (0, 0)),
                   pl.BlockSpec((1, DIM), lambda i: (0, 0))],
        out_shape=[jax.ShapeDtypeStruct((N_EDGES, DIM), jnp.float32),
                   jax.ShapeDtypeStruct((1, DIM), jnp.float32),
                   jax.ShapeDtypeStruct((1, DIM), jnp.float32)],
    )(u, s, t, wt, b)


def _bnrelu_split_kernel(z_ref, s_ref, t_ref, lo_ref, hi_ref):
    y = jnp.maximum(z_ref[...] * s_ref[...] + t_ref[...], 0.0)
    lo_ref[...] = y[:, :_HALF]
    hi_ref[...] = y[:, _HALF:]


def _bnrelu_split(z, s, t):
    nb = N_EDGES // _BN_EDGE
    return pl.pallas_call(
        _bnrelu_split_kernel,
        grid=(nb,),
        in_specs=[pl.BlockSpec((_BN_EDGE, DIM), lambda i: (i, 0)),
                  pl.BlockSpec((1, DIM), lambda i: (0, 0)),
                  pl.BlockSpec((1, DIM), lambda i: (0, 0))],
        out_specs=[pl.BlockSpec((_BN_EDGE, _HALF), lambda i: (i, 0))] * 2,
        out_shape=[jax.ShapeDtypeStruct((N_EDGES, _HALF), jnp.float32)] * 2,
    )(z, s, t)


def _final_kernel(x_ref, dfl, dfh, dol, doh, wt_ref, b_ref, g_ref, bb_ref,
                  out_ref):
    d = jnp.concatenate([dfl[...] + dol[...], dfh[...] + doh[...]], axis=1)
    xn = x_ref[...] + d
    y = jnp.dot(xn, wt_ref[...], preferred_element_type=jnp.float32)
    y = y + b_ref[...]
    m = jnp.mean(y, axis=0, keepdims=True)
    v = jnp.mean((y - m) ** 2, axis=0, keepdims=True)
    out_ref[...] = jnp.maximum(
        (y - m) / jnp.sqrt(v + EPS) * g_ref[...] + bb_ref[...], 0.0)


def _final_stage(x, d_parts, W, b, g, bb):
    dfl, dfh, dol, doh = d_parts
    return pl.pallas_call(
        _final_kernel,
        out_shape=jax.ShapeDtypeStruct((N_NODES, DIM), jnp.float32),
    )(x, dfl[:N_NODES], dfh[:N_NODES], dol[:N_NODES], doh[:N_NODES],
      W.T, b.reshape(1, DIM), g.reshape(1, DIM), bb.reshape(1, DIM))


def _affine(sum_, sumsq, g, b, n):
    m = sum_ / n
    v = sumsq / n - m * m
    s = g.reshape(1, DIM) / jnp.sqrt(v + EPS)
    t = b.reshape(1, DIM) - m * s
    return s, t


def kernel(x, edges, params):
    p = params
    src = jnp.asarray(edges[:, 0], jnp.int32)
    dst = jnp.asarray(edges[:, 1], jnp.int32)

    fia, fib, foa, fob = _node_feats(
        x, p['FI_fc1a_W'].T, p['FI_fc1b_W'].T, p['FO_fc1a_W'].T,
        p['FO_fc1b_W'].T, p['FI_fc1b_b'].reshape(1, DIM),
        p['FO_fc1b_b'].reshape(1, DIM))

    fi_in, s_fi, q_fi = _gather_add(fia, fib, src, dst)
    fo_in, s_fo, q_fo = _gather_add(foa, fob, src, dst)

    s1i, t1i = _affine(s_fi, q_fi, p['FI_bn1_g'], p['FI_bn1_b'], N_EDGES)
    s1o, t1o = _affine(s_fo, q_fo, p['FO_bn1_g'], p['FO_bn1_b'], N_EDGES)

    z_fi, sz_fi, qz_fi = _bnmm(fi_in, s1i, t1i, p['FI_fc2_W'].T,
                               p['FI_fc2_b'].reshape(1, DIM))
    s2i, t2i = _affine(sz_fi, qz_fi, p['FI_bn2_g'], p['FI_bn2_b'], N_EDGES)
    yfi_lo, yfi_hi = _bnrelu_split(z_fi, s2i, t2i)
    dfi_lo, dfi_hi = _segsum(yfi_lo, yfi_hi, dst)

    z_fo, sz_fo, qz_fo = _bnmm(fo_in, s1o, t1o, p['FO_fc2_W'].T,
                               p['FO_fc2_b'].reshape(1, DIM))
    s2o, t2o = _affine(sz_fo, qz_fo, p['FO_bn2_g'], p['FO_bn2_b'], N_EDGES)
    yfo_lo, yfo_hi = _bnrelu_split(z_fo, s2o, t2o)
    dfo_lo, dfo_hi = _segsum(yfo_lo, yfo_hi, src)

    return _final_stage(x, (dfi_lo, dfi_hi, dfo_lo, dfo_hi),
                        p['FP_fc_W'], p['FP_fc_b'], p['FP_bn_g'], p['FP_bn_b'])
